# baseline (speedup 1.0000x reference)
import jax
import jax.numpy as jnp
from jax.experimental import pallas as pl

_B, _J, _D, _AE, _A = 16, 4096, 32, 16, 512
_HH, _EH, _L = 64, 128, 32
_R = _B * _J
_EPS = 1e-5

_A_W1, _A_B1, _A_G1, _A_BE1 = 0, 49, 50, 51
_A_WM2, _A_BM2, _A_GM2, _A_BEM2 = 52, 180, 181, 182
_B_W2, _B_B2, _B_G2, _B_BE2 = 0, 64, 65, 66
_C_WM1, _C_BM1, _C_GM1, _C_BEM1 = 0, 32, 33, 34


def _col(row):
    return jnp.transpose(row)


def _fused_kernel(xr_ref, mi_ref, feT_ref, ae_ref, gA_ref, gB_ref, gC_ref,
                  out_ref):
    W1 = gA_ref[_A_W1:_A_W1 + 49, :]
    W1c = W1 - jnp.mean(W1, axis=1, keepdims=True)
    b1row = gA_ref[_A_B1:_A_B1 + 1, :]
    b1c_col = _col(b1row - jnp.mean(b1row))
    g1_col = _col(gA_ref[_A_G1:_A_G1 + 1, :])
    be1_col = _col(gA_ref[_A_BE1:_A_BE1 + 1, :])
    wc_row = W1c[0:1, :]
    wc_col = _col(wc_row)
    wcg_col = wc_col * g1_col
    c1 = jnp.mean(wc_row * wc_row).reshape(1, 1)
    W1fT = jnp.transpose(W1c[1:1 + _D, :])
    W1aT = jnp.transpose(W1c[1 + _D:49, :])

    W2 = gB_ref[_B_W2:_B_W2 + _HH, :]
    W2c = W2 - jnp.mean(W2, axis=1, keepdims=True)
    W2cT = jnp.transpose(W2c).astype(jnp.bfloat16)
    b2row = gB_ref[_B_B2:_B_B2 + 1, :]
    b2c_col = _col(b2row - jnp.mean(b2row))
    g2_col = _col(gB_ref[_B_G2:_B_G2 + 1, :])
    be2_col = _col(gB_ref[_B_BE2:_B_BE2 + 1, :])

    mi = mi_ref[...]
    idx = mi[:, _R:]
    onehotT = (jax.lax.broadcasted_iota(jnp.int32, (_A, _J), 0) == idx
               ).astype(jnp.float32)
    aeT = jax.lax.dot_general(
        ae_ref[...], onehotT, (((0,), (0,)), ((), ())),
        preferred_element_type=jnp.float32)

    uc = (jnp.dot(W1fT, feT_ref[...],
                  preferred_element_type=jnp.float32)
          + jnp.dot(W1aT, aeT, preferred_element_type=jnp.float32)
          + b1c_col)
    ucg = (uc * g1_col).astype(jnp.bfloat16)
    A = jnp.mean(uc * uc, axis=0, keepdims=True)
    Bq = 2.0 * jnp.mean(uc * wc_col, axis=0, keepdims=True)

    xr = xr_ref[...]
    A_t = jnp.tile(A, (1, _B))
    B_t = jnp.tile(Bq, (1, _B))
    var1 = A_t + xr * (B_t + xr * c1)
    rs = jax.lax.rsqrt(var1 + _EPS)
    ucg_t = jnp.tile(ucg, (1, _B))
    h1 = jnp.maximum(ucg_t.astype(jnp.float32) * rs
                     + wcg_col * (xr * rs) + be1_col, 0.0
                     ).astype(jnp.bfloat16)

    pre2 = jnp.dot(W2cT, h1,
                   preferred_element_type=jnp.float32) + b2c_col
    rs2 = jax.lax.rsqrt(jnp.mean(pre2 * pre2, axis=0, keepdims=True) + _EPS)
    h2 = jnp.maximum(pre2 * (rs2 * g2_col) + be2_col, 0.0)

    mrf = mi[:, :_R].astype(jnp.float32)
    masked = jnp.concatenate([h2 * mrf, mrf], axis=0)

    seg = (jax.lax.broadcasted_iota(jnp.int32, (_R, _B), 0) // _J
           == jax.lax.broadcasted_iota(jnp.int32, (_R, _B), 1)
           ).astype(jnp.float32)
    acc = jnp.dot(masked, seg,
                  preferred_element_type=jnp.float32)

    cnt = acc[_D:_D + 1, :]
    c = jnp.where(cnt > 0,
                  acc[:_D, :] / jnp.maximum(cnt, 1.0), 0.0)

    Wm1 = gC_ref[_C_WM1:_C_WM1 + _D, :]
    Wm1c = Wm1 - jnp.mean(Wm1, axis=1, keepdims=True)
    bm1row = gC_ref[_C_BM1:_C_BM1 + 1, :]
    bm1c_col = _col(bm1row - jnp.mean(bm1row))
    gm1_col = _col(gC_ref[_C_GM1:_C_GM1 + 1, :])
    bem1_col = _col(gC_ref[_C_BEM1:_C_BEM1 + 1, :])
    Wm2 = gA_ref[_A_WM2:_A_WM2 + _EH, :]
    Wm2c = Wm2 - jnp.mean(Wm2, axis=1, keepdims=True)
    bm2row = gA_ref[_A_BM2:_A_BM2 + 1, :]
    bm2c_col = _col(bm2row - jnp.mean(bm2row))
    gm2_col = _col(gA_ref[_A_GM2:_A_GM2 + 1, :])
    bem2_col = _col(gA_ref[_A_BEM2:_A_BEM2 + 1, :])

    p1 = jnp.dot(jnp.transpose(Wm1c), c,
                 preferred_element_type=jnp.float32) + bm1c_col
    r1 = jax.lax.rsqrt(jnp.mean(p1 * p1, axis=0, keepdims=True) + _EPS)
    t1 = jnp.maximum(p1 * (r1 * gm1_col) + bem1_col, 0.0)
    p2 = jnp.dot(jnp.transpose(Wm2c), t1,
                 preferred_element_type=jnp.float32) + bm2c_col
    r2 = jax.lax.rsqrt(jnp.mean(p2 * p2, axis=0, keepdims=True) + _EPS)
    t2 = jnp.maximum(p2 * (r2 * gm2_col) + bem2_col, 0.0)
    out_ref[...] = t2


def kernel(x, mask, feature_embedding, atse_embedding, atse_index_per_j,
           W1, b1, g1, be1, W2, b2, g2, be2,
           Wm1, bm1, gm1, bem1, Wm2, bm2, gm2, bem2):
    xr = x.reshape(1, _R)
    mi = jnp.concatenate([mask.reshape(1, _R),
                          atse_index_per_j.reshape(1, _J)], axis=1)
    feT = feature_embedding.T

    gA = jnp.concatenate([
        W1, b1.reshape(1, -1), g1.reshape(1, -1), be1.reshape(1, -1),
        Wm2, bm2.reshape(1, -1), gm2.reshape(1, -1), bem2.reshape(1, -1),
    ], axis=0)
    gB = jnp.concatenate([
        W2, b2.reshape(1, -1), g2.reshape(1, -1), be2.reshape(1, -1),
    ], axis=0)
    gC = jnp.concatenate([
        Wm1, bm1.reshape(1, -1), gm1.reshape(1, -1), bem1.reshape(1, -1),
    ], axis=0)

    out = pl.pallas_call(
        _fused_kernel,
        out_shape=jax.ShapeDtypeStruct((2 * _L, _B), jnp.float32),
    )(xr, mi, feT, atse_embedding, gA, gB, gC)
    outT = out.T
    return outT[:, :_L], outT[:, _L:]
